# Initial kernel scaffold; baseline (speedup 1.0000x reference)
import jax
import jax.numpy as jnp
from jax.experimental import pallas as pl


def kernel(x):
    raise NotImplementedError("write your pallas kernel here")


# speedup vs baseline: 2.1789x; 2.1789x over previous
import functools

import jax
import jax.numpy as jnp
from jax import lax
from jax.experimental import pallas as pl
from jax.experimental.pallas import tpu as pltpu
from jax.experimental.pallas import tpu_sc as plsc

ROWS = 128
COLS = 32768
LANES = 16
NV = COLS // LANES
NWORKERS = 32
ROWS_PER_W = ROWS // NWORKERS
RADIX = 256
NBITS = 8
NPASSES = 4

_SIGN = jnp.int32(-2147483648)


def _to_sortable(k):
    return k ^ ((k >> 31) | _SIGN)


def _from_sortable(k):
    return k ^ ((~(k >> 31)) | _SIGN)


def _sort_body(x_hbm, out_hbm, buf_a, buf_b, hist):
    cid = lax.axis_index("c")
    sid = lax.axis_index("s")
    wid = sid * 2 + cid

    ones = jnp.ones((LANES,), jnp.int32)
    zeros = jnp.zeros((LANES,), jnp.int32)

    def do_row(r, carry):
        row = wid * ROWS_PER_W + r
        pltpu.sync_copy(x_hbm.at[row], buf_a)

        for p in range(NPASSES):
            src, dst = (buf_a, buf_b) if p % 2 == 0 else (buf_b, buf_a)
            shift = p * NBITS

            for h in range(RADIX // LANES):
                hist[pl.ds(h * LANES, LANES)] = zeros

            def hist_step(i, c, _src=src, _shift=shift, _p=p):
                k = _src[pl.ds(i * LANES, LANES)]
                if _p == 0:
                    k = _to_sortable(k)
                d = (k >> _shift) & (RADIX - 1)
                plsc.addupdate_scatter(hist, (d,), ones)
                return c

            lax.fori_loop(0, NV, hist_step, 0)

            base = jnp.int32(0)
            for h in range(RADIX // LANES):
                v = hist[pl.ds(h * LANES, LANES)]
                inc = plsc.cumsum(v)
                hist[pl.ds(h * LANES, LANES)] = inc - v + base
                base = base + jnp.sum(v)

            def perm_step(i, c, _src=src, _dst=dst, _shift=shift, _p=p):
                k = _src[pl.ds(i * LANES, LANES)]
                if _p == 0:
                    k = _to_sortable(k)
                d = (k >> _shift) & (RADIX - 1)
                occ, _ = plsc.scan_count(d)
                pos = plsc.load_gather(hist, (d,)) + occ - 1
                if _p == NPASSES - 1:
                    k = _from_sortable(k)
                plsc.store_scatter(_dst, (pos,), k)
                plsc.addupdate_scatter(hist, (d,), ones)
                return c

            lax.fori_loop(0, NV, perm_step, 0)

        pltpu.sync_copy(buf_a, out_hbm.at[row])
        return carry

    lax.fori_loop(0, ROWS_PER_W, do_row, 0)


@jax.jit
def kernel(x):
    xi = lax.bitcast_convert_type(x, jnp.int32)
    run = pl.kernel(
        _sort_body,
        out_type=jax.ShapeDtypeStruct((ROWS, COLS), jnp.int32),
        mesh=plsc.VectorSubcoreMesh(core_axis_name="c", subcore_axis_name="s"),
        compiler_params=pltpu.CompilerParams(needs_layout_passes=False),
        scratch_types=[
            pltpu.VMEM((COLS,), jnp.int32),
            pltpu.VMEM((COLS,), jnp.int32),
            pltpu.VMEM((RADIX,), jnp.int32),
        ],
    )
    return lax.bitcast_convert_type(run(xi), jnp.float32)


# speedup vs baseline: 5.3882x; 2.4729x over previous
import functools

import jax
import jax.numpy as jnp
from jax import lax
from jax.experimental import pallas as pl
from jax.experimental.pallas import tpu as pltpu
from jax.experimental.pallas import tpu_sc as plsc

ROWS = 128
COLS = 32768
LANES = 16
NV = COLS // LANES
NWORKERS = 32
ROWS_PER_W = ROWS // NWORKERS
RADIX = 256
NBITS = 8
NPASSES = 4
K = 4
SV = NV // K
CH = COLS // K
CHB = 13

_SIGN = jnp.int32(-2147483648)


def _to_sortable(k):
    return k ^ ((k >> 31) | _SIGN)


def _from_sortable(k):
    return k ^ ((~(k >> 31)) | _SIGN)


def _sort_body(x_hbm, out_hbm, buf_a, buf_b, o0, o1, o2, o3, histn):
    cid = lax.axis_index("c")
    sid = lax.axis_index("s")
    wid = sid * 2 + cid
    offs = (o0, o1, o2, o3)

    ones = jnp.ones((LANES,), jnp.int32)
    zeros = jnp.zeros((LANES,), jnp.int32)

    def do_row(r, carry):
        row = wid * ROWS_PER_W + r
        pltpu.sync_copy(x_hbm.at[row], buf_a)

        for h in range(K * RADIX // LANES):
            histn[pl.ds(h * LANES, LANES)] = zeros

        def h0_step(i, c):
            for k in range(K):
                kv = _to_sortable(buf_a[pl.ds(i * LANES + k * CH, LANES)])
                d = kv & (RADIX - 1)
                plsc.addupdate_scatter(histn, (d + k * RADIX,), ones)
            return c

        lax.fori_loop(0, SV, h0_step, 0)

        for p in range(NPASSES):
            src, dst = (buf_a, buf_b) if p % 2 == 0 else (buf_b, buf_a)
            shift = p * NBITS

            def mk_offs(h, base):
                t = [histn[pl.ds(h * LANES + k * RADIX, LANES)] for k in range(K)]
                tot = (t[0] + t[1]) + (t[2] + t[3])
                inc = plsc.cumsum(tot)
                run = inc - tot + (base - 1)
                for k in range(K):
                    offs[k][pl.ds(h * LANES, LANES)] = run
                    run = run + t[k]
                    histn[pl.ds(h * LANES + k * RADIX, LANES)] = zeros
                return base + jnp.sum(tot)

            lax.fori_loop(0, RADIX // LANES, mk_offs, jnp.int32(0))

            def perm_step(i, c, _src=src, _dst=dst, _shift=shift, _p=p):
                kvs, dss, poss = [], [], []
                for k in range(K):
                    kv = _src[pl.ds(i * LANES + k * CH, LANES)]
                    if _p == 0:
                        kv = _to_sortable(kv)
                    d = (kv >> _shift) & (RADIX - 1)
                    occ, _ = plsc.scan_count(d)
                    pos = plsc.load_gather(offs[k], (d,)) + occ
                    kvs.append(kv)
                    dss.append(d)
                    poss.append(pos)
                for k in range(K):
                    out_v = kvs[k]
                    if _p == NPASSES - 1:
                        out_v = _from_sortable(out_v)
                    plsc.store_scatter(_dst, (poss[k],), out_v)
                    plsc.addupdate_scatter(offs[k], (dss[k],), ones)
                    if _p < NPASSES - 1:
                        dn = (kvs[k] >> (_shift + NBITS)) & (RADIX - 1)
                        hidx = ((poss[k] >> CHB) << NBITS) + dn
                        plsc.addupdate_scatter(histn, (hidx,), ones)
                return c

            lax.fori_loop(0, SV, perm_step, 0)

        pltpu.sync_copy(buf_a, out_hbm.at[row])
        return carry

    lax.fori_loop(0, ROWS_PER_W, do_row, 0)


@jax.jit
def kernel(x):
    xi = lax.bitcast_convert_type(x, jnp.int32)
    run = pl.kernel(
        _sort_body,
        out_type=jax.ShapeDtypeStruct((ROWS, COLS), jnp.int32),
        mesh=plsc.VectorSubcoreMesh(core_axis_name="c", subcore_axis_name="s"),
        compiler_params=pltpu.CompilerParams(needs_layout_passes=False),
        scratch_types=[
            pltpu.VMEM((COLS,), jnp.int32),
            pltpu.VMEM((COLS,), jnp.int32),
            pltpu.VMEM((RADIX,), jnp.int32),
            pltpu.VMEM((RADIX,), jnp.int32),
            pltpu.VMEM((RADIX,), jnp.int32),
            pltpu.VMEM((RADIX,), jnp.int32),
            pltpu.VMEM((K * RADIX,), jnp.int32),
        ],
    )
    return lax.bitcast_convert_type(run(xi), jnp.float32)


# speedup vs baseline: 6.3685x; 1.1819x over previous
import functools

import jax
import jax.numpy as jnp
from jax import lax
from jax.experimental import pallas as pl
from jax.experimental.pallas import tpu as pltpu
from jax.experimental.pallas import tpu_sc as plsc

ROWS = 128
COLS = 32768
LANES = 16
NV = COLS // LANES
NWORKERS = 32
ROWS_PER_W = ROWS // NWORKERS
RADIX = 2048
SHIFTS = (0, 11, 22)
MASKS = (2047, 2047, 1023)
NPASSES = 3
K = 4
SV = NV // K
CH = COLS // K
CHB = 13
RB = 11

_SIGN = jnp.int32(-2147483648)


def _to_sortable(k):
    return k ^ ((k >> 31) | _SIGN)


def _from_sortable(k):
    return k ^ ((~(k >> 31)) | _SIGN)


def _sort_body(x_hbm, out_hbm, buf_a, buf_b, o0, o1, o2, o3, histn):
    cid = lax.axis_index("c")
    sid = lax.axis_index("s")
    wid = sid * 2 + cid
    offs = (o0, o1, o2, o3)

    ones = jnp.ones((LANES,), jnp.int32)
    zeros = jnp.zeros((LANES,), jnp.int32)

    def do_row(r, carry):
        row = wid * ROWS_PER_W + r
        pltpu.sync_copy(x_hbm.at[row], buf_a)

        def zero_step(h, c):
            histn[pl.ds(h * LANES, LANES)] = zeros
            return c

        lax.fori_loop(0, K * RADIX // LANES, zero_step, 0)

        def h0_step(i, c):
            for k in range(K):
                kv = _to_sortable(buf_a[pl.ds(i * LANES + k * CH, LANES)])
                d = kv & MASKS[0]
                plsc.addupdate_scatter(histn, (d + k * RADIX,), ones)
            return c

        lax.fori_loop(0, SV, h0_step, 0)

        for p in range(NPASSES):
            src, dst = (buf_a, buf_b) if p % 2 == 0 else (buf_b, buf_a)
            shift = SHIFTS[p]
            mask = MASKS[p]

            def mk_offs(h, base):
                t = [histn[pl.ds(h * LANES + k * RADIX, LANES)] for k in range(K)]
                tot = (t[0] + t[1]) + (t[2] + t[3])
                inc = plsc.cumsum(tot)
                run = inc - tot + (base - 1)
                for k in range(K):
                    offs[k][pl.ds(h * LANES, LANES)] = run
                    run = run + t[k]
                    histn[pl.ds(h * LANES + k * RADIX, LANES)] = zeros
                return base + jnp.sum(tot)

            lax.fori_loop(0, RADIX // LANES, mk_offs, jnp.int32(0))

            def perm_step(i, c, _src=src, _dst=dst, _shift=shift, _mask=mask, _p=p):
                kvs, dss, poss = [], [], []
                for k in range(K):
                    kv = _src[pl.ds(i * LANES + k * CH, LANES)]
                    if _p == 0:
                        kv = _to_sortable(kv)
                    d = (kv >> _shift) & _mask
                    occ, _ = plsc.scan_count(d)
                    pos = plsc.load_gather(offs[k], (d,)) + occ
                    kvs.append(kv)
                    dss.append(d)
                    poss.append(pos)
                for k in range(K):
                    out_v = kvs[k]
                    if _p == NPASSES - 1:
                        out_v = _from_sortable(out_v)
                    plsc.store_scatter(_dst, (poss[k],), out_v)
                    plsc.addupdate_scatter(offs[k], (dss[k],), ones)
                    if _p < NPASSES - 1:
                        dn = (kvs[k] >> SHIFTS[_p + 1]) & MASKS[_p + 1]
                        hidx = ((poss[k] >> CHB) << RB) + dn
                        plsc.addupdate_scatter(histn, (hidx,), ones)
                return c

            lax.fori_loop(0, SV, perm_step, 0)

        pltpu.sync_copy(buf_b, out_hbm.at[row])
        return carry

    lax.fori_loop(0, ROWS_PER_W, do_row, 0)


@jax.jit
def kernel(x):
    xi = lax.bitcast_convert_type(x, jnp.int32)
    run = pl.kernel(
        _sort_body,
        out_type=jax.ShapeDtypeStruct((ROWS, COLS), jnp.int32),
        mesh=plsc.VectorSubcoreMesh(core_axis_name="c", subcore_axis_name="s"),
        compiler_params=pltpu.CompilerParams(needs_layout_passes=False),
        scratch_types=[
            pltpu.VMEM((COLS,), jnp.int32),
            pltpu.VMEM((COLS,), jnp.int32),
            pltpu.VMEM((RADIX,), jnp.int32),
            pltpu.VMEM((RADIX,), jnp.int32),
            pltpu.VMEM((RADIX,), jnp.int32),
            pltpu.VMEM((RADIX,), jnp.int32),
            pltpu.VMEM((K * RADIX,), jnp.int32),
        ],
    )
    return lax.bitcast_convert_type(run(xi), jnp.float32)
